# baseline (speedup 1.0000x reference)
import jax
import jax.numpy as jnp
from jax.experimental import pallas as pl
from jax.experimental.pallas import tpu as pltpu

TEXT_DIM = 512
AUDIO_DIM = 128
H = 128
OUT = 2 * H
T = 2048
BATCH = 8
WINDOW = 20
HOPS = 3
AIM = 256
NC = 7

BT = 256
NB = T // BT
BTA = 256
NBA = T // BTA
GBT = 256
NGB = T // GBT
PAD = 32


def _xw_kernel(xt_ref, xa_ref, Wt_ref, Wa_ref, b_ref, outf_ref, outb_ref):
    xt = xt_ref[...].reshape(BTA * BATCH, TEXT_DIM)
    xa = xa_ref[...].reshape(BTA * BATCH, AUDIO_DIM)
    g = jnp.dot(xt, Wt_ref[...], preferred_element_type=jnp.float32)
    g = g + jnp.dot(xa, Wa_ref[...], preferred_element_type=jnp.float32)
    g = g + b_ref[...]
    g = g.reshape(BTA, BATCH, 8 * H)
    outf_ref[...] = g[:, :, : 4 * H]
    outb_ref[...] = g[:, :, 4 * H:]


def _lstm_kernel(xwf_ref, xwb_ref, Whh_ref, hf_ref, hb_ref, h_sc, c_sc):
    i = pl.program_id(0)

    @pl.when(i == 0)
    def _():
        h_sc[...] = jnp.zeros_like(h_sc)
        c_sc[...] = jnp.zeros_like(c_sc)

    def gates(g, c):
        ii = jax.nn.sigmoid(g[:, 0:H])
        ff = jax.nn.sigmoid(g[:, H:2 * H])
        uu = jnp.tanh(g[:, 2 * H:3 * H])
        oo = jax.nn.sigmoid(g[:, 3 * H:4 * H])
        cn = ff * c + ii * uu
        hn = oo * jnp.tanh(cn)
        return hn, cn

    def body(k, carry):
        h, c = carry
        g = jnp.dot(h.astype(jnp.bfloat16), Whh_ref[...],
                    preferred_element_type=jnp.float32)
        gf = g[:, : 4 * H] + xwf_ref[k]
        gb = g[:, 4 * H:] + xwb_ref[BT - 1 - k]
        hf_new, cf_new = gates(gf, c[:, :H])
        hb_new, cb_new = gates(gb, c[:, H:])
        hf_ref[k] = hf_new
        hb_ref[BT - 1 - k] = hb_new
        return (jnp.concatenate([hf_new, hb_new], axis=1),
                jnp.concatenate([cf_new, cb_new], axis=1))

    h, c = jax.lax.fori_loop(0, BT, body, (h_sc[...], c_sc[...]), unroll=8)
    h_sc[...] = h
    c_sc[...] = c


def _grn_aim_kernel(hf_ref, hb_ref, WgL_ref, WgG_ref, Wx_ref, Wgr_ref, Wc_ref,
                    bg_ref, bfu_ref, bc_ref, out_ref, ext_ref):
    lstm = jnp.concatenate([hf_ref[...], hb_ref[...]], axis=1)

    r = jax.lax.broadcasted_iota(jnp.int32, (GBT, GBT + PAD), 0)
    e = jax.lax.broadcasted_iota(jnp.int32, (GBT, GBT + PAD), 1)
    Mb = jnp.where((e >= r + (PAD - WINDOW)) & (e <= r + PAD), 1.0, 0.0)

    t_idx = jax.lax.broadcasted_iota(jnp.int32, (T, 1), 0)
    inv_deg = 1.0 / jnp.minimum(t_idx + 1, WINDOW + 1).astype(jnp.float32)

    ext_ref[0:PAD, :] = jnp.zeros((PAD, OUT), jnp.float32)
    cur = lstm
    acc = lstm
    for _ in range(HOPS):
        ext_ref[PAD:PAD + T, :] = cur
        blocks = []
        for blk in range(NGB):
            seg = ext_ref[blk * GBT: blk * GBT + GBT + PAD, :]
            blocks.append(jnp.dot(Mb, seg, preferred_element_type=jnp.float32))
        ws = jnp.concatenate(blocks, axis=0)
        cur = ws * inv_deg
        acc = acc + cur
    graph = acc * (1.0 / (HOPS + 1))

    gate = jax.nn.sigmoid(
        jnp.dot(lstm, WgL_ref[...], preferred_element_type=jnp.float32)
        + jnp.dot(graph, WgG_ref[...], preferred_element_type=jnp.float32)
        + bg_ref[...])
    fused = jnp.tanh(
        gate * jnp.dot(lstm, Wx_ref[...], preferred_element_type=jnp.float32)
        + (1.0 - gate) * jnp.dot(graph, Wgr_ref[...], preferred_element_type=jnp.float32)
        + bfu_ref[...])
    out_ref[0] = jnp.dot(fused, Wc_ref[...], preferred_element_type=jnp.float32) + bc_ref[...]


def kernel(text_embeds, audio_feats, speaker_ids, W_ih_f, W_hh_f, b_f,
           W_ih_b, W_hh_b, b_b, Wg, bg, Wx, Wgr, bf, Wc, bc):
    f32 = jnp.float32
    xt = jnp.swapaxes(text_embeds, 0, 1)
    xa = jnp.swapaxes(audio_feats, 0, 1)
    Wt = jnp.concatenate([W_ih_f[:, :TEXT_DIM], W_ih_b[:, :TEXT_DIM]], axis=0).T
    Wa = jnp.concatenate([W_ih_f[:, TEXT_DIM:], W_ih_b[:, TEXT_DIM:]], axis=0).T
    bcat = jnp.concatenate([b_f, b_b]).reshape(1, 8 * H)
    Whh = jnp.zeros((2 * H, 8 * H), f32)
    Whh = Whh.at[:H, :4 * H].set(W_hh_f.T)
    Whh = Whh.at[H:, 4 * H:].set(W_hh_b.T).astype(jnp.bfloat16)

    xwf, xwb = pl.pallas_call(
        _xw_kernel,
        grid=(NBA,),
        in_specs=[
            pl.BlockSpec((BTA, BATCH, TEXT_DIM), lambda i: (i, 0, 0)),
            pl.BlockSpec((BTA, BATCH, AUDIO_DIM), lambda i: (i, 0, 0)),
            pl.BlockSpec((TEXT_DIM, 8 * H), lambda i: (0, 0)),
            pl.BlockSpec((AUDIO_DIM, 8 * H), lambda i: (0, 0)),
            pl.BlockSpec((1, 8 * H), lambda i: (0, 0)),
        ],
        out_specs=[
            pl.BlockSpec((BTA, BATCH, 4 * H), lambda i: (i, 0, 0)),
            pl.BlockSpec((BTA, BATCH, 4 * H), lambda i: (i, 0, 0)),
        ],
        out_shape=[
            jax.ShapeDtypeStruct((T, BATCH, 4 * H), f32),
            jax.ShapeDtypeStruct((T, BATCH, 4 * H), f32),
        ],
    )(xt, xa, Wt, Wa, bcat)

    hf, hb = pl.pallas_call(
        _lstm_kernel,
        grid=(NB,),
        in_specs=[
            pl.BlockSpec((BT, BATCH, 4 * H), lambda i: (i, 0, 0)),
            pl.BlockSpec((BT, BATCH, 4 * H), lambda i: (NB - 1 - i, 0, 0)),
            pl.BlockSpec((2 * H, 8 * H), lambda i: (0, 0)),
        ],
        out_specs=[
            pl.BlockSpec((BT, BATCH, H), lambda i: (i, 0, 0)),
            pl.BlockSpec((BT, BATCH, H), lambda i: (NB - 1 - i, 0, 0)),
        ],
        out_shape=[
            jax.ShapeDtypeStruct((T, BATCH, H), f32),
            jax.ShapeDtypeStruct((T, BATCH, H), f32),
        ],
        scratch_shapes=[
            pltpu.VMEM((BATCH, 2 * H), f32),
            pltpu.VMEM((BATCH, 2 * H), f32),
        ],
    )(xwf, xwb, Whh)

    hf2 = hf.reshape(T, BATCH * H)
    hb2 = hb.reshape(T, BATCH * H)
    WgL = Wg[:, :OUT].T
    WgG = Wg[:, OUT:].T
    WxT = Wx.T
    WgrT = Wgr.T
    WcT = Wc.T
    bg2 = bg.reshape(1, AIM)
    bf2 = bf.reshape(1, AIM)
    bc2 = bc.reshape(1, NC)

    logits = pl.pallas_call(
        _grn_aim_kernel,
        grid=(BATCH,),
        in_specs=[
            pl.BlockSpec((T, H), lambda b: (0, b)),
            pl.BlockSpec((T, H), lambda b: (0, b)),
            pl.BlockSpec((OUT, AIM), lambda b: (0, 0)),
            pl.BlockSpec((OUT, AIM), lambda b: (0, 0)),
            pl.BlockSpec((OUT, AIM), lambda b: (0, 0)),
            pl.BlockSpec((OUT, AIM), lambda b: (0, 0)),
            pl.BlockSpec((AIM, NC), lambda b: (0, 0)),
            pl.BlockSpec((1, AIM), lambda b: (0, 0)),
            pl.BlockSpec((1, AIM), lambda b: (0, 0)),
            pl.BlockSpec((1, NC), lambda b: (0, 0)),
        ],
        out_specs=pl.BlockSpec((1, T, NC), lambda b: (b, 0, 0)),
        out_shape=jax.ShapeDtypeStruct((BATCH, T, NC), f32),
        scratch_shapes=[pltpu.VMEM((T + PAD, OUT), f32)],
    )(hf2, hb2, WgL, WgG, WxT, WgrT, WcT, bg2, bf2, bc2)

    return logits


# speedup vs baseline: 1.2553x; 1.2553x over previous
import jax
import jax.numpy as jnp
from jax.experimental import pallas as pl
from jax.experimental.pallas import tpu as pltpu

TEXT_DIM = 512
AUDIO_DIM = 128
H = 128
OUT = 2 * H
T = 2048
BATCH = 8
WINDOW = 20
HOPS = 3
AIM = 256
NC = 7

BT = 256
NB = T // BT
BTA = 256
NBA = T // BTA
GBT = 256
NGB = T // GBT
PAD = 32


def _xw_kernel(xt_ref, xa_ref, Wt_ref, Wa_ref, b_ref, outf_ref, outb_ref):
    for b in range(BATCH):
        g = jnp.dot(xt_ref[b], Wt_ref[...], preferred_element_type=jnp.float32)
        g = g + jnp.dot(xa_ref[b], Wa_ref[...], preferred_element_type=jnp.float32)
        g = g + b_ref[...]
        outf_ref[:, b, :] = g[:, : 4 * H]
        outb_ref[:, b, :] = g[:, 4 * H:]


def _lstm_kernel(xwf_ref, xwb_ref, Whf_ref, Whb_ref, hf_ref, hb_ref,
                 hf_sc, hb_sc, cf_sc, cb_sc):
    i = pl.program_id(0)

    @pl.when(i == 0)
    def _():
        hf_sc[...] = jnp.zeros_like(hf_sc)
        hb_sc[...] = jnp.zeros_like(hb_sc)
        cf_sc[...] = jnp.zeros_like(cf_sc)
        cb_sc[...] = jnp.zeros_like(cb_sc)

    def gates(g, c):
        ii = jax.nn.sigmoid(g[:, 0:H])
        ff = jax.nn.sigmoid(g[:, H:2 * H])
        uu = jnp.tanh(g[:, 2 * H:3 * H])
        oo = jax.nn.sigmoid(g[:, 3 * H:4 * H])
        cn = ff * c + ii * uu
        hn = oo * jnp.tanh(cn)
        return hn, cn

    def body(k, carry):
        h_f, h_b, c_f, c_b = carry
        gf = jnp.dot(h_f.astype(jnp.bfloat16), Whf_ref[...],
                     preferred_element_type=jnp.float32) + xwf_ref[k]
        gb = jnp.dot(h_b.astype(jnp.bfloat16), Whb_ref[...],
                     preferred_element_type=jnp.float32) + xwb_ref[BT - 1 - k]
        hf_new, cf_new = gates(gf, c_f)
        hb_new, cb_new = gates(gb, c_b)
        hf_ref[k] = hf_new
        hb_ref[BT - 1 - k] = hb_new
        return (hf_new, hb_new, cf_new, cb_new)

    carry = (hf_sc[...], hb_sc[...], cf_sc[...], cb_sc[...])
    h_f, h_b, c_f, c_b = jax.lax.fori_loop(0, BT, body, carry, unroll=8)
    hf_sc[...] = h_f
    hb_sc[...] = h_b
    cf_sc[...] = c_f
    cb_sc[...] = c_b


def _grn_aim_kernel(hf_ref, hb_ref, WgL_ref, WgG_ref, Wx_ref, Wgr_ref, Wc_ref,
                    bg_ref, bfu_ref, bc_ref, out_ref, ext_ref):
    lstm = jnp.concatenate([hf_ref[...], hb_ref[...]], axis=1)

    r = jax.lax.broadcasted_iota(jnp.int32, (GBT, GBT + PAD), 0)
    e = jax.lax.broadcasted_iota(jnp.int32, (GBT, GBT + PAD), 1)
    Mb = jnp.where((e >= r + (PAD - WINDOW)) & (e <= r + PAD), 1.0, 0.0)

    t_idx = jax.lax.broadcasted_iota(jnp.int32, (T, 1), 0)
    inv_deg = 1.0 / jnp.minimum(t_idx + 1, WINDOW + 1).astype(jnp.float32)

    ext_ref[0:PAD, :] = jnp.zeros((PAD, OUT), jnp.float32)
    cur = lstm
    acc = lstm
    for _ in range(HOPS):
        ext_ref[PAD:PAD + T, :] = cur
        blocks = []
        for blk in range(NGB):
            seg = ext_ref[blk * GBT: blk * GBT + GBT + PAD, :]
            blocks.append(jnp.dot(Mb, seg, preferred_element_type=jnp.float32))
        ws = jnp.concatenate(blocks, axis=0)
        cur = ws * inv_deg
        acc = acc + cur
    graph = acc * (1.0 / (HOPS + 1))

    gate = jax.nn.sigmoid(
        jnp.dot(lstm, WgL_ref[...], preferred_element_type=jnp.float32)
        + jnp.dot(graph, WgG_ref[...], preferred_element_type=jnp.float32)
        + bg_ref[...])
    fused = jnp.tanh(
        gate * jnp.dot(lstm, Wx_ref[...], preferred_element_type=jnp.float32)
        + (1.0 - gate) * jnp.dot(graph, Wgr_ref[...], preferred_element_type=jnp.float32)
        + bfu_ref[...])
    out_ref[0] = jnp.dot(fused, Wc_ref[...], preferred_element_type=jnp.float32) + bc_ref[...]


def kernel(text_embeds, audio_feats, speaker_ids, W_ih_f, W_hh_f, b_f,
           W_ih_b, W_hh_b, b_b, Wg, bg, Wx, Wgr, bf, Wc, bc):
    f32 = jnp.float32
    Wt = jnp.concatenate([W_ih_f[:, :TEXT_DIM], W_ih_b[:, :TEXT_DIM]], axis=0).T
    Wa = jnp.concatenate([W_ih_f[:, TEXT_DIM:], W_ih_b[:, TEXT_DIM:]], axis=0).T
    bcat = jnp.concatenate([b_f, b_b]).reshape(1, 8 * H)
    Whf = W_hh_f.T.astype(jnp.bfloat16)
    Whb = W_hh_b.T.astype(jnp.bfloat16)

    xwf, xwb = pl.pallas_call(
        _xw_kernel,
        grid=(NBA,),
        in_specs=[
            pl.BlockSpec((BATCH, BTA, TEXT_DIM), lambda i: (0, i, 0)),
            pl.BlockSpec((BATCH, BTA, AUDIO_DIM), lambda i: (0, i, 0)),
            pl.BlockSpec((TEXT_DIM, 8 * H), lambda i: (0, 0)),
            pl.BlockSpec((AUDIO_DIM, 8 * H), lambda i: (0, 0)),
            pl.BlockSpec((1, 8 * H), lambda i: (0, 0)),
        ],
        out_specs=[
            pl.BlockSpec((BTA, BATCH, 4 * H), lambda i: (i, 0, 0)),
            pl.BlockSpec((BTA, BATCH, 4 * H), lambda i: (i, 0, 0)),
        ],
        out_shape=[
            jax.ShapeDtypeStruct((T, BATCH, 4 * H), f32),
            jax.ShapeDtypeStruct((T, BATCH, 4 * H), f32),
        ],
    )(text_embeds, audio_feats, Wt, Wa, bcat)

    hf, hb = pl.pallas_call(
        _lstm_kernel,
        grid=(NB,),
        in_specs=[
            pl.BlockSpec((BT, BATCH, 4 * H), lambda i: (i, 0, 0)),
            pl.BlockSpec((BT, BATCH, 4 * H), lambda i: (NB - 1 - i, 0, 0)),
            pl.BlockSpec((H, 4 * H), lambda i: (0, 0)),
            pl.BlockSpec((H, 4 * H), lambda i: (0, 0)),
        ],
        out_specs=[
            pl.BlockSpec((BT, BATCH, H), lambda i: (i, 0, 0)),
            pl.BlockSpec((BT, BATCH, H), lambda i: (NB - 1 - i, 0, 0)),
        ],
        out_shape=[
            jax.ShapeDtypeStruct((T, BATCH, H), f32),
            jax.ShapeDtypeStruct((T, BATCH, H), f32),
        ],
        scratch_shapes=[
            pltpu.VMEM((BATCH, H), f32),
            pltpu.VMEM((BATCH, H), f32),
            pltpu.VMEM((BATCH, H), f32),
            pltpu.VMEM((BATCH, H), f32),
        ],
    )(xwf, xwb, Whf, Whb)

    hf2 = hf.reshape(T, BATCH * H)
    hb2 = hb.reshape(T, BATCH * H)
    WgL = Wg[:, :OUT].T
    WgG = Wg[:, OUT:].T
    WxT = Wx.T
    WgrT = Wgr.T
    WcT = Wc.T
    bg2 = bg.reshape(1, AIM)
    bf2 = bf.reshape(1, AIM)
    bc2 = bc.reshape(1, NC)

    logits = pl.pallas_call(
        _grn_aim_kernel,
        grid=(BATCH,),
        in_specs=[
            pl.BlockSpec((T, H), lambda b: (0, b)),
            pl.BlockSpec((T, H), lambda b: (0, b)),
            pl.BlockSpec((OUT, AIM), lambda b: (0, 0)),
            pl.BlockSpec((OUT, AIM), lambda b: (0, 0)),
            pl.BlockSpec((OUT, AIM), lambda b: (0, 0)),
            pl.BlockSpec((OUT, AIM), lambda b: (0, 0)),
            pl.BlockSpec((AIM, NC), lambda b: (0, 0)),
            pl.BlockSpec((1, AIM), lambda b: (0, 0)),
            pl.BlockSpec((1, AIM), lambda b: (0, 0)),
            pl.BlockSpec((1, NC), lambda b: (0, 0)),
        ],
        out_specs=pl.BlockSpec((1, T, NC), lambda b: (b, 0, 0)),
        out_shape=jax.ShapeDtypeStruct((BATCH, T, NC), f32),
        scratch_shapes=[pltpu.VMEM((T + PAD, OUT), f32)],
    )(hf2, hb2, WgL, WgG, WxT, WgrT, WcT, bg2, bf2, bc2)

    return logits


# speedup vs baseline: 1.2986x; 1.0345x over previous
import jax
import jax.numpy as jnp
from jax.experimental import pallas as pl
from jax.experimental.pallas import tpu as pltpu

TEXT_DIM = 512
AUDIO_DIM = 128
H = 128
OUT = 2 * H
T = 2048
BATCH = 8
WINDOW = 20
HOPS = 3
AIM = 256
NC = 7

BT = 256
NB = T // BT
BTA = 256
NBA = T // BTA
GBT = 256
NGB = T // GBT
PAD = 32


def _xw_kernel(xt_ref, xa_ref, Wt_ref, Wa_ref, b_ref, outf_ref, outb_ref):
    for b in range(BATCH):
        g = jnp.dot(xt_ref[b], Wt_ref[...], preferred_element_type=jnp.float32)
        g = g + jnp.dot(xa_ref[b], Wa_ref[...], preferred_element_type=jnp.float32)
        g = g + b_ref[...]
        outf_ref[:, b, :] = g[:, : 4 * H]
        outb_ref[:, b, :] = g[:, 4 * H:]


def _lstm_kernel(xwf_ref, xwb_ref, Whf_ref, Whb_ref, hf_ref, hb_ref,
                 hf_sc, hb_sc, cf_sc, cb_sc):
    i = pl.program_id(0)

    @pl.when(i == 0)
    def _():
        hf_sc[...] = jnp.zeros_like(hf_sc)
        hb_sc[...] = jnp.zeros_like(hb_sc)
        cf_sc[...] = jnp.zeros_like(cf_sc)
        cb_sc[...] = jnp.zeros_like(cb_sc)

    def sig(x):
        return 0.5 * jnp.tanh(0.5 * x) + 0.5

    def gates(g, c):
        ii = sig(g[:, 0:H])
        ff = sig(g[:, H:2 * H])
        uu = jnp.tanh(g[:, 2 * H:3 * H])
        oo = sig(g[:, 3 * H:4 * H])
        cn = ff * c + ii * uu
        hn = oo * jnp.tanh(cn)
        return hn, cn

    def body(k, carry):
        h_f, h_b, c_f, c_b = carry
        gf = jnp.dot(h_f.astype(jnp.bfloat16), Whf_ref[...],
                     preferred_element_type=jnp.float32) + xwf_ref[k]
        gb = jnp.dot(h_b.astype(jnp.bfloat16), Whb_ref[...],
                     preferred_element_type=jnp.float32) + xwb_ref[BT - 1 - k]
        hf_new, cf_new = gates(gf, c_f)
        hb_new, cb_new = gates(gb, c_b)
        hf_ref[k] = hf_new
        hb_ref[BT - 1 - k] = hb_new
        return (hf_new, hb_new, cf_new, cb_new)

    carry = (hf_sc[...], hb_sc[...], cf_sc[...], cb_sc[...])
    h_f, h_b, c_f, c_b = jax.lax.fori_loop(0, BT, body, carry, unroll=16)
    hf_sc[...] = h_f
    hb_sc[...] = h_b
    cf_sc[...] = c_f
    cb_sc[...] = c_b


def _grn_aim_kernel(hf_ref, hb_ref, WgL_ref, WgG_ref, Wx_ref, Wgr_ref, Wc_ref,
                    bg_ref, bfu_ref, bc_ref, out_ref, ext_ref):
    lstm = jnp.concatenate([hf_ref[...], hb_ref[...]], axis=1)

    r = jax.lax.broadcasted_iota(jnp.int32, (GBT, GBT + PAD), 0)
    e = jax.lax.broadcasted_iota(jnp.int32, (GBT, GBT + PAD), 1)
    Mb = jnp.where((e >= r + (PAD - WINDOW)) & (e <= r + PAD), 1.0, 0.0)

    t_idx = jax.lax.broadcasted_iota(jnp.int32, (T, 1), 0)
    inv_deg = 1.0 / jnp.minimum(t_idx + 1, WINDOW + 1).astype(jnp.float32)

    ext_ref[0:PAD, :] = jnp.zeros((PAD, OUT), jnp.float32)
    cur = lstm
    acc = lstm
    for _ in range(HOPS):
        ext_ref[PAD:PAD + T, :] = cur
        blocks = []
        for blk in range(NGB):
            seg = ext_ref[blk * GBT: blk * GBT + GBT + PAD, :]
            blocks.append(jnp.dot(Mb, seg, preferred_element_type=jnp.float32))
        ws = jnp.concatenate(blocks, axis=0)
        cur = ws * inv_deg
        acc = acc + cur
    graph = acc * (1.0 / (HOPS + 1))

    gate = jax.nn.sigmoid(
        jnp.dot(lstm, WgL_ref[...], preferred_element_type=jnp.float32)
        + jnp.dot(graph, WgG_ref[...], preferred_element_type=jnp.float32)
        + bg_ref[...])
    fused = jnp.tanh(
        gate * jnp.dot(lstm, Wx_ref[...], preferred_element_type=jnp.float32)
        + (1.0 - gate) * jnp.dot(graph, Wgr_ref[...], preferred_element_type=jnp.float32)
        + bfu_ref[...])
    out_ref[0] = jnp.dot(fused, Wc_ref[...], preferred_element_type=jnp.float32) + bc_ref[...]


def kernel(text_embeds, audio_feats, speaker_ids, W_ih_f, W_hh_f, b_f,
           W_ih_b, W_hh_b, b_b, Wg, bg, Wx, Wgr, bf, Wc, bc):
    f32 = jnp.float32
    Wt = jnp.concatenate([W_ih_f[:, :TEXT_DIM], W_ih_b[:, :TEXT_DIM]], axis=0).T
    Wa = jnp.concatenate([W_ih_f[:, TEXT_DIM:], W_ih_b[:, TEXT_DIM:]], axis=0).T
    bcat = jnp.concatenate([b_f, b_b]).reshape(1, 8 * H)
    Whf = W_hh_f.T.astype(jnp.bfloat16)
    Whb = W_hh_b.T.astype(jnp.bfloat16)

    xwf, xwb = pl.pallas_call(
        _xw_kernel,
        grid=(NBA,),
        in_specs=[
            pl.BlockSpec((BATCH, BTA, TEXT_DIM), lambda i: (0, i, 0)),
            pl.BlockSpec((BATCH, BTA, AUDIO_DIM), lambda i: (0, i, 0)),
            pl.BlockSpec((TEXT_DIM, 8 * H), lambda i: (0, 0)),
            pl.BlockSpec((AUDIO_DIM, 8 * H), lambda i: (0, 0)),
            pl.BlockSpec((1, 8 * H), lambda i: (0, 0)),
        ],
        out_specs=[
            pl.BlockSpec((BTA, BATCH, 4 * H), lambda i: (i, 0, 0)),
            pl.BlockSpec((BTA, BATCH, 4 * H), lambda i: (i, 0, 0)),
        ],
        out_shape=[
            jax.ShapeDtypeStruct((T, BATCH, 4 * H), f32),
            jax.ShapeDtypeStruct((T, BATCH, 4 * H), f32),
        ],
    )(text_embeds, audio_feats, Wt, Wa, bcat)

    hf, hb = pl.pallas_call(
        _lstm_kernel,
        grid=(NB,),
        in_specs=[
            pl.BlockSpec((BT, BATCH, 4 * H), lambda i: (i, 0, 0)),
            pl.BlockSpec((BT, BATCH, 4 * H), lambda i: (NB - 1 - i, 0, 0)),
            pl.BlockSpec((H, 4 * H), lambda i: (0, 0)),
            pl.BlockSpec((H, 4 * H), lambda i: (0, 0)),
        ],
        out_specs=[
            pl.BlockSpec((BT, BATCH, H), lambda i: (i, 0, 0)),
            pl.BlockSpec((BT, BATCH, H), lambda i: (NB - 1 - i, 0, 0)),
        ],
        out_shape=[
            jax.ShapeDtypeStruct((T, BATCH, H), f32),
            jax.ShapeDtypeStruct((T, BATCH, H), f32),
        ],
        scratch_shapes=[
            pltpu.VMEM((BATCH, H), f32),
            pltpu.VMEM((BATCH, H), f32),
            pltpu.VMEM((BATCH, H), f32),
            pltpu.VMEM((BATCH, H), f32),
        ],
    )(xwf, xwb, Whf, Whb)

    hf2 = hf.reshape(T, BATCH * H)
    hb2 = hb.reshape(T, BATCH * H)
    WgL = Wg[:, :OUT].T
    WgG = Wg[:, OUT:].T
    WxT = Wx.T
    WgrT = Wgr.T
    WcT = Wc.T
    bg2 = bg.reshape(1, AIM)
    bf2 = bf.reshape(1, AIM)
    bc2 = bc.reshape(1, NC)

    logits = pl.pallas_call(
        _grn_aim_kernel,
        grid=(BATCH,),
        in_specs=[
            pl.BlockSpec((T, H), lambda b: (0, b)),
            pl.BlockSpec((T, H), lambda b: (0, b)),
            pl.BlockSpec((OUT, AIM), lambda b: (0, 0)),
            pl.BlockSpec((OUT, AIM), lambda b: (0, 0)),
            pl.BlockSpec((OUT, AIM), lambda b: (0, 0)),
            pl.BlockSpec((OUT, AIM), lambda b: (0, 0)),
            pl.BlockSpec((AIM, NC), lambda b: (0, 0)),
            pl.BlockSpec((1, AIM), lambda b: (0, 0)),
            pl.BlockSpec((1, AIM), lambda b: (0, 0)),
            pl.BlockSpec((1, NC), lambda b: (0, 0)),
        ],
        out_specs=pl.BlockSpec((1, T, NC), lambda b: (b, 0, 0)),
        out_shape=jax.ShapeDtypeStruct((BATCH, T, NC), f32),
        scratch_shapes=[pltpu.VMEM((T + PAD, OUT), f32)],
    )(hf2, hb2, WgL, WgG, WxT, WgrT, WcT, bg2, bf2, bc2)

    return logits
